# baseline (speedup 1.0000x reference)
import functools

import jax
import jax.numpy as jnp
from jax import lax
from jax.experimental import pallas as pl
from jax.experimental.pallas import tpu as pltpu
from jax.experimental.pallas import tpu_sc as plsc

DIM = 32
LANES = 16
CHUNK = 128


@functools.cache
def _build_sc_kernel(rows):
    info = plsc.get_sparse_core_info()
    nc, ns = info.num_cores, info.num_subcores
    nw = nc * ns
    per_w = rows // nw
    n_chunks = per_w // CHUNK
    assert rows == nw * n_chunks * CHUNK

    mesh = plsc.VectorSubcoreMesh(core_axis_name="c", subcore_axis_name="s")

    @functools.partial(
        pl.kernel,
        mesh=mesh,
        out_type=jax.ShapeDtypeStruct((rows, DIM), jnp.float32),
        compiler_params=pltpu.CompilerParams(use_tc_tiling_on_sc=False),
        scratch_types=[
            pltpu.VMEM((n_chunks, CHUNK), jnp.int32),
            pltpu.VMEM((n_chunks, CHUNK), jnp.float32),
            pltpu.VMEM((CHUNK, DIM), jnp.float32),
            pltpu.VMEM((CHUNK, DIM), jnp.float32),
            pltpu.SemaphoreType.DMA,
        ],
    )
    def body(idx_hbm, inp_hbm, emb_hbm, bias_hbm, out_hbm,
             idx_v, inp_v, emb_v, bias_v, sem):
        wid = lax.axis_index("s") * nc + lax.axis_index("c")
        c0 = wid * n_chunks

        pltpu.sync_copy(idx_hbm.at[wid], idx_v)
        pltpu.sync_copy(inp_hbm.at[wid], inp_v)

        vec_per_chunk = CHUNK // LANES

        def mask_body(q, _):
            j = q // vec_per_chunk
            k = (q % vec_per_chunk) * LANES
            v = inp_v[j, pl.ds(k, LANES)]
            isnan = v != v
            inp_v[j, pl.ds(k, LANES)] = jnp.where(isnan, jnp.float32(0.0), v)
            ii = idx_v[j, pl.ds(k, LANES)]
            idx_v[j, pl.ds(k, LANES)] = jnp.where(isnan, jnp.int32(0), ii)
            return 0

        lax.fori_loop(0, n_chunks * vec_per_chunk, mask_body, 0)

        def chunk_body(j, _):
            ce = pltpu.async_copy(emb_hbm.at[idx_v.at[j]], emb_v, sem)
            cb = pltpu.async_copy(bias_hbm.at[idx_v.at[j]], bias_v, sem)
            ce.wait()
            cb.wait()

            def group_body(g, _):
                sv = inp_v[j, pl.ds(g * LANES, LANES)]
                for i in range(LANES):
                    r = g * LANES + i
                    s = lax.gather(
                        sv, jnp.full((LANES, 1), i, jnp.int32),
                        lax.GatherDimensionNumbers(
                            offset_dims=(), collapsed_slice_dims=(0,),
                            start_index_map=(0,)),
                        (1,), mode=lax.GatherScatterMode.PROMISE_IN_BOUNDS)
                    for h in range(DIM // LANES):
                        sl = pl.ds(h * LANES, LANES)
                        x = emb_v[r, sl] * s + bias_v[r, sl]
                        e = jnp.exp(jnp.minimum(x, jnp.float32(0.0))) - jnp.float32(1.0)
                        emb_v[r, sl] = jnp.where(x > jnp.float32(0.0), x, e)
                return 0

            lax.fori_loop(0, CHUNK // LANES, group_body, 0)
            pltpu.sync_copy(emb_v, out_hbm.at[pl.ds((c0 + j) * CHUNK, CHUNK)])
            return 0

        lax.fori_loop(0, n_chunks, chunk_body, 0)

    def call(input_tokens, inputs, embeddings, bias):
        idx = input_tokens.astype(jnp.int32).reshape(nw, n_chunks, CHUNK)
        inp = inputs.reshape(nw, n_chunks, CHUNK)
        return body(idx, inp, embeddings, bias)

    return call


def kernel(input_tokens, inputs, embeddings, bias):
    b, l = input_tokens.shape
    rows = b * l
    out = _build_sc_kernel(rows)(input_tokens, inputs, embeddings, bias)
    return out.reshape(b, l, DIM)


# speedup vs baseline: 1.6075x; 1.6075x over previous
import functools

import jax
import jax.numpy as jnp
from jax import lax
from jax.experimental import pallas as pl
from jax.experimental.pallas import tpu as pltpu
from jax.experimental.pallas import tpu_sc as plsc

DIM = 32
LANES = 16
BLK = 128
TSTRIP = 4096
TBLOCK = 8192
KW = 16


@functools.cache
def _build(b, l, vocab):
    info = plsc.get_sparse_core_info()
    nc, ns = info.num_cores, info.num_subcores
    nw = nc * ns
    assert b == nw * BLK, (b, nw)

    n_full = vocab // BLK
    tail = vocab - n_full * BLK
    per_w = n_full // nw
    n_extra = n_full - per_w * nw
    n_blk = per_w // KW
    krem = per_w - n_blk * KW
    fused_n = TBLOCK * (n_full + (1 if tail else 0))

    mesh = plsc.VectorSubcoreMesh(core_axis_name="c", subcore_axis_name="s")

    @functools.partial(
        pl.kernel,
        mesh=mesh,
        out_type=jax.ShapeDtypeStruct((fused_n // TSTRIP, DIM, BLK), jnp.float32),
        scratch_types=[
            pltpu.VMEM((DIM, KW * BLK), jnp.float32),
            pltpu.SemaphoreType.DMA,
        ],
    )
    def repack(emb_hbm, bias_hbm, tail_hbm, fused_hbm, buf_v, sem):
        wid = lax.axis_index("s") * nc + lax.axis_index("c")
        c_base = wid * per_w
        fused_3d = fused_hbm

        def round_body(tbl, soff, c, k):
            pltpu.sync_copy(tbl.at[:, pl.ds(c * BLK, k * BLK)],
                            buf_v.at[:, pl.ds(0, k * BLK)])
            copies = []
            for j in range(k):
                copies.append(pltpu.async_copy(
                    buf_v.at[:, pl.ds(j * BLK, BLK)],
                    fused_3d.at[(c + j) * 2 + soff],
                    sem))
            for cp in copies:
                cp.wait()

        for tbl, soff in ((emb_hbm, 0), (bias_hbm, 1)):
            def blk_body(i, _):
                round_body(tbl, soff, c_base + i * KW, KW)
                return 0

            lax.fori_loop(0, n_blk, blk_body, 0)
            if krem:
                round_body(tbl, soff, c_base + n_blk * KW, krem)

        @pl.when(wid < n_extra)
        def _():
            ce = per_w * nw + wid
            for tbl, soff in ((emb_hbm, 0), (bias_hbm, 1)):
                round_body(tbl, soff, ce, 1)

        if tail:
            @pl.when(wid == n_extra)
            def _():
                for soff in (0, 1):
                    pltpu.sync_copy(tail_hbm.at[soff],
                                    fused_3d.at[n_full * 2 + soff])

    @functools.partial(
        pl.kernel,
        mesh=mesh,
        out_type=jax.ShapeDtypeStruct((l, DIM, b), jnp.float32),
        scratch_types=[
            pltpu.VMEM((l, BLK), jnp.int32),
            pltpu.VMEM((l, BLK), jnp.float32),
            pltpu.VMEM((2, DIM, BLK), jnp.int32),
            pltpu.VMEM((2, DIM, BLK), jnp.float32),
            pltpu.VMEM((2, DIM, BLK), jnp.float32),
            pltpu.SemaphoreType.DMA,
            pltpu.SemaphoreType.DMA,
            pltpu.SemaphoreType.DMA,
        ],
    )
    def lookup(idx_hbm, inp_hbm, fused_hbm, out_hbm,
               idx_v, inp_v, eidx_v, gemb_v, gbias_v, sem0, sem1, sem_out):
        wid = lax.axis_index("s") * nc + lax.axis_index("c")
        b0 = wid * BLK
        emb_flat = fused_hbm.at[0, 0]
        bias_flat = fused_hbm.at[1, 0]

        pltpu.sync_copy(idx_hbm.at[:, pl.ds(b0, BLK)], idx_v)
        pltpu.sync_copy(inp_hbm.at[:, pl.ds(b0, BLK)], inp_v)

        vec_per_blk = BLK // LANES

        def mask_body(q, _):
            j = q // vec_per_blk
            k = (q % vec_per_blk) * LANES
            v = inp_v[j, pl.ds(k, LANES)]
            isnan = v != v
            inp_v[j, pl.ds(k, LANES)] = jnp.where(isnan, jnp.float32(0.0), v)
            ii = idx_v[j, pl.ds(k, LANES)]
            idx_v[j, pl.ds(k, LANES)] = jnp.where(isnan, jnp.int32(0), ii)
            return 0

        lax.fori_loop(0, l * vec_per_blk, mask_body, 0)

        sems = (sem0, sem1)

        def build_and_fire(cl, buf):
            def g_body(g, _):
                sl = pl.ds(g * LANES, LANES)
                t16 = idx_v[cl, sl]
                base = (lax.shift_left(lax.shift_right_logical(t16, 7), 13)
                        + jnp.bitwise_and(t16, jnp.int32(BLK - 1)))
                for d in range(DIM):
                    eidx_v[buf, d, sl] = base + jnp.int32(d * BLK)
                return 0

            lax.fori_loop(0, vec_per_blk, g_body, 0)
            for d in range(DIM):
                pltpu.async_copy(emb_flat.at[eidx_v.at[buf, d]],
                                 gemb_v.at[buf, d], sems[buf])
                pltpu.async_copy(bias_flat.at[eidx_v.at[buf, d]],
                                 gbias_v.at[buf, d], sems[buf])

        def drain(buf):
            for d in range(DIM):
                pltpu.make_async_copy(emb_flat.at[eidx_v.at[buf, d]],
                                      gemb_v.at[buf, d], sems[buf]).wait()
                pltpu.make_async_copy(bias_flat.at[eidx_v.at[buf, d]],
                                      gbias_v.at[buf, d], sems[buf]).wait()

        def compute_and_store(cl, buf):
            def cg_body(g, _):
                sl = pl.ds(g * LANES, LANES)
                s = inp_v[cl, sl]
                for d in range(DIM):
                    x = gemb_v[buf, d, sl] * s + gbias_v[buf, d, sl]
                    e = jnp.exp(jnp.minimum(x, jnp.float32(0.0))) - jnp.float32(1.0)
                    gemb_v[buf, d, sl] = jnp.where(x > jnp.float32(0.0), x, e)
                return 0

            lax.fori_loop(0, vec_per_blk, cg_body, 0)
            pltpu.async_copy(gemb_v.at[buf],
                             out_hbm.at[cl, :, pl.ds(b0, BLK)], sem_out).wait()

        def chunk_body(j, _):
            build_and_fire(j, 0)
            drain(0)
            compute_and_store(j, 0)
            return 0

        lax.fori_loop(0, l, chunk_body, 0)

    def call(input_tokens, inputs, embeddings, bias):
        idx_t = jnp.transpose(input_tokens)
        inp_t = jnp.transpose(inputs, (1, 2, 0)).reshape(l, b)
        emb_t = jnp.transpose(embeddings)
        bias_t = jnp.transpose(bias)
        n_full = vocab // BLK
        tail = vocab - n_full * BLK
        if tail:
            pad = ((0, 0), (0, BLK - tail))
            tail_blk = jnp.stack([
                jnp.pad(jnp.transpose(embeddings[n_full * BLK:, :]), pad),
                jnp.pad(jnp.transpose(bias[n_full * BLK:, :]), pad),
            ])
        else:
            tail_blk = jnp.zeros((2, DIM, BLK), jnp.float32)
        fused = repack(emb_t, bias_t, tail_blk)
        out = lookup(idx_t, inp_t, fused)
        return jnp.transpose(out, (2, 0, 1))

    return call


def kernel(input_tokens, inputs, embeddings, bias):
    b, l = input_tokens.shape
    vocab = embeddings.shape[0]
    return _build(b, l, vocab)(input_tokens, inputs, embeddings, bias)


# speedup vs baseline: 1.7715x; 1.1020x over previous
import functools

import jax
import jax.numpy as jnp
from jax import lax
from jax.experimental import pallas as pl
from jax.experimental.pallas import tpu as pltpu
from jax.experimental.pallas import tpu_sc as plsc

DIM = 32
LANES = 16
BLK = 128
TSTRIP = 4096
TBLOCK = 8192
KW = 16


@functools.cache
def _build(b, l, vocab):
    info = plsc.get_sparse_core_info()
    nc, ns = info.num_cores, info.num_subcores
    nw = nc * ns
    assert b == nw * BLK, (b, nw)

    n_full = vocab // BLK
    tail = vocab - n_full * BLK
    per_w = n_full // nw
    n_extra = n_full - per_w * nw
    n_blk = per_w // KW
    krem = per_w - n_blk * KW
    fused_n = TBLOCK * (n_full + (1 if tail else 0))

    mesh = plsc.VectorSubcoreMesh(core_axis_name="c", subcore_axis_name="s")

    @functools.partial(
        pl.kernel,
        mesh=mesh,
        out_type=jax.ShapeDtypeStruct((fused_n // TSTRIP, DIM, BLK), jnp.float32),
        scratch_types=[
            pltpu.VMEM((DIM, KW * BLK), jnp.float32),
            pltpu.SemaphoreType.DMA,
        ],
    )
    def repack(emb_hbm, bias_hbm, tail_hbm, fused_hbm, buf_v, sem):
        wid = lax.axis_index("s") * nc + lax.axis_index("c")
        c_base = wid * per_w
        fused_3d = fused_hbm

        def round_body(tbl, soff, c, k):
            pltpu.sync_copy(tbl.at[:, pl.ds(c * BLK, k * BLK)],
                            buf_v.at[:, pl.ds(0, k * BLK)])
            copies = []
            for j in range(k):
                copies.append(pltpu.async_copy(
                    buf_v.at[:, pl.ds(j * BLK, BLK)],
                    fused_3d.at[(c + j) * 2 + soff],
                    sem))
            for cp in copies:
                cp.wait()

        for tbl, soff in ((emb_hbm, 0), (bias_hbm, 1)):
            def blk_body(i, _):
                round_body(tbl, soff, c_base + i * KW, KW)
                return 0

            lax.fori_loop(0, n_blk, blk_body, 0)
            if krem:
                round_body(tbl, soff, c_base + n_blk * KW, krem)

        @pl.when(wid < n_extra)
        def _():
            ce = per_w * nw + wid
            for tbl, soff in ((emb_hbm, 0), (bias_hbm, 1)):
                round_body(tbl, soff, ce, 1)

        if tail:
            @pl.when(wid == n_extra)
            def _():
                for soff in (0, 1):
                    pltpu.sync_copy(tail_hbm.at[soff],
                                    fused_3d.at[n_full * 2 + soff])

    @functools.partial(
        pl.kernel,
        mesh=mesh,
        out_type=jax.ShapeDtypeStruct((l, DIM, b), jnp.float32),
        scratch_types=[
            pltpu.VMEM((l, BLK), jnp.int32),
            pltpu.VMEM((l, BLK), jnp.float32),
            pltpu.VMEM((2, DIM, BLK), jnp.int32),
            pltpu.VMEM((2, DIM, BLK), jnp.float32),
            pltpu.VMEM((2, DIM, BLK), jnp.float32),
            pltpu.SemaphoreType.DMA,
            pltpu.SemaphoreType.DMA,
            pltpu.SemaphoreType.DMA,
        ],
    )
    def lookup(idx_hbm, inp_hbm, fused_hbm, out_hbm,
               idx_v, inp_v, eidx_v, gemb_v, gbias_v, sem0, sem1, sem_out):
        wid = lax.axis_index("s") * nc + lax.axis_index("c")
        b0 = wid * BLK
        emb_flat = fused_hbm.at[0, 0]
        bias_flat = fused_hbm.at[1, 0]

        pltpu.sync_copy(idx_hbm.at[:, pl.ds(b0, BLK)], idx_v)
        pltpu.sync_copy(inp_hbm.at[:, pl.ds(b0, BLK)], inp_v)

        vec_per_blk = BLK // LANES

        def mask_body(q, _):
            j = q // vec_per_blk
            k = (q % vec_per_blk) * LANES
            v = inp_v[j, pl.ds(k, LANES)]
            isnan = v != v
            inp_v[j, pl.ds(k, LANES)] = jnp.where(isnan, jnp.float32(0.0), v)
            ii = idx_v[j, pl.ds(k, LANES)]
            idx_v[j, pl.ds(k, LANES)] = jnp.where(isnan, jnp.int32(0), ii)
            return 0

        lax.fori_loop(0, l * vec_per_blk, mask_body, 0)

        sems = (sem0, sem1)

        def build_and_fire(cl, buf):
            def g_body(g, _):
                sl = pl.ds(g * LANES, LANES)
                t16 = idx_v[cl, sl]
                base = (lax.shift_left(lax.shift_right_logical(t16, 7), 13)
                        + jnp.bitwise_and(t16, jnp.int32(BLK - 1)))
                for d in range(DIM):
                    eidx_v[buf, d, sl] = base + jnp.int32(d * BLK)
                return 0

            lax.fori_loop(0, vec_per_blk, g_body, 0)
            for d in range(DIM):
                pltpu.async_copy(emb_flat.at[eidx_v.at[buf, d]],
                                 gemb_v.at[buf, d], sems[buf])
                pltpu.async_copy(bias_flat.at[eidx_v.at[buf, d]],
                                 gbias_v.at[buf, d], sems[buf])

        def drain(buf):
            for d in range(DIM):
                pltpu.make_async_copy(emb_flat.at[eidx_v.at[buf, d]],
                                      gemb_v.at[buf, d], sems[buf]).wait()
                pltpu.make_async_copy(bias_flat.at[eidx_v.at[buf, d]],
                                      gbias_v.at[buf, d], sems[buf]).wait()

        def compute_and_store(cl, buf):
            def cg_body(g, _):
                sl = pl.ds(g * LANES, LANES)
                s = inp_v[cl, sl]
                for d in range(DIM):
                    x = gemb_v[buf, d, sl] * s + gbias_v[buf, d, sl]
                    e = jnp.exp(jnp.minimum(x, jnp.float32(0.0))) - jnp.float32(1.0)
                    gemb_v[buf, d, sl] = jnp.where(x > jnp.float32(0.0), x, e)
                return 0

            lax.fori_loop(0, vec_per_blk, cg_body, 0)
            pltpu.async_copy(gemb_v.at[buf],
                             out_hbm.at[cl, :, pl.ds(b0, BLK)], sem_out).wait()

        build_and_fire(0, 0)
        n_pairs = l // 2

        def chunk_body(j, _):
            build_and_fire(2 * j + 1, 1)
            drain(0)
            compute_and_store(2 * j, 0)

            @pl.when(2 * j + 2 < l)
            def _():
                build_and_fire(2 * j + 2, 0)

            drain(1)
            compute_and_store(2 * j + 1, 1)
            return 0

        lax.fori_loop(0, n_pairs, chunk_body, 0)
        if l % 2:
            drain(0)
            compute_and_store(l - 1, 0)

    def call(input_tokens, inputs, embeddings, bias):
        idx_t = jnp.transpose(input_tokens)
        inp_t = jnp.transpose(inputs, (1, 2, 0)).reshape(l, b)
        emb_t = jnp.transpose(embeddings)
        bias_t = jnp.transpose(bias)
        n_full = vocab // BLK
        tail = vocab - n_full * BLK
        if tail:
            pad = ((0, 0), (0, BLK - tail))
            tail_blk = jnp.stack([
                jnp.pad(jnp.transpose(embeddings[n_full * BLK:, :]), pad),
                jnp.pad(jnp.transpose(bias[n_full * BLK:, :]), pad),
            ])
        else:
            tail_blk = jnp.zeros((2, DIM, BLK), jnp.float32)
        fused = repack(emb_t, bias_t, tail_blk)
        out = lookup(idx_t, inp_t, fused)
        return jnp.transpose(out, (2, 0, 1))

    return call


def kernel(input_tokens, inputs, embeddings, bias):
    b, l = input_tokens.shape
    vocab = embeddings.shape[0]
    return _build(b, l, vocab)(input_tokens, inputs, embeddings, bias)
